# baseline (speedup 1.0000x reference)
import functools

import jax
import jax.numpy as jnp
from jax import lax
from jax.experimental import pallas as pl
from jax.experimental.pallas import tpu as pltpu
from jax.experimental.pallas import tpu_sc as plsc

N = 10000
E = 320000
D = 128
NC = 2
NS = 16
NW = NC * NS
B = 80
CHUNKS = 125
EPW = CHUNKS * B
NPAD = 10112
RPT = NPAD // NS
RPT_LAST = N - (NS - 1) * RPT
DW = 16

_mesh = plsc.VectorSubcoreMesh(core_axis_name="c", subcore_axis_name="s")


@functools.partial(
    pl.kernel,
    mesh=_mesh,
    out_type=jax.ShapeDtypeStruct((NC, N, DW), jnp.float32),
    scratch_types=[
        pltpu.VMEM((CHUNKS, B), jnp.int32),
        pltpu.VMEM((B, DW), jnp.float32),
        pltpu.VMEM_SHARED((NPAD, DW), jnp.float32),
        pltpu.SemaphoreType.DMA,
        pltpu.SemaphoreType.DMA,
    ],
)
def _deg_kernel(cols_hbm, ones_hbm, zeros_hbm, out_hbm, colv, onesv, acc,
                ssem0, ssem1):
    ssem = (ssem0, ssem1)
    c = lax.axis_index("c")
    s = lax.axis_index("s")
    w = c * NS + s
    pltpu.sync_copy(cols_hbm.at[w], colv)
    pltpu.sync_copy(ones_hbm, onesv)
    base = s * RPT
    pltpu.sync_copy(zeros_hbm.at[pl.ds(base, RPT)], acc.at[pl.ds(base, RPT)])
    plsc.subcore_barrier()

    def body(j, b):
        @pl.when(j >= 2)
        def _():
            pltpu.make_async_copy(onesv, acc.at[colv.at[j - 2]], ssem[b]).wait()

        pltpu.async_copy(onesv, acc.at[colv.at[j]], ssem[b], add=True)

    def outer(i, carry):
        body(2 * i, 0)
        body(2 * i + 1, 1)
        return carry

    lax.fori_loop(0, (CHUNKS - 1) // 2, outer, 0)
    body(CHUNKS - 1, (CHUNKS - 1) % 2)
    pltpu.make_async_copy(onesv, acc.at[colv.at[CHUNKS - 2]],
                          ssem[(CHUNKS - 2) % 2]).wait()
    pltpu.make_async_copy(onesv, acc.at[colv.at[CHUNKS - 1]],
                          ssem[(CHUNKS - 1) % 2]).wait()
    plsc.subcore_barrier()

    @pl.when(s < NS - 1)
    def _():
        pltpu.sync_copy(acc.at[pl.ds(base, RPT)], out_hbm.at[c, pl.ds(base, RPT)])

    @pl.when(s == NS - 1)
    def _():
        pltpu.sync_copy(acc.at[pl.ds(base, RPT_LAST)],
                        out_hbm.at[c, pl.ds(base, RPT_LAST)])


@functools.partial(
    pl.kernel,
    mesh=_mesh,
    out_type=jax.ShapeDtypeStruct((NC, N, D), jnp.float32),
    scratch_types=[
        pltpu.VMEM((CHUNKS * B,), jnp.int32),
        pltpu.VMEM((CHUNKS, B), jnp.int32),
        pltpu.VMEM_SHARED((NPAD, D), jnp.float32),
        pltpu.VMEM((B, D), jnp.float32),
        pltpu.VMEM((B, D), jnp.float32),
        pltpu.SemaphoreType.DMA,
        pltpu.SemaphoreType.DMA,
        pltpu.SemaphoreType.DMA,
        pltpu.SemaphoreType.DMA,
    ],
)
def _agg_kernel(g_hbm, rows_hbm, cols_hbm, zeros_hbm, out_hbm,
                rowsv, colsv, acc, gbuf0, gbuf1,
                gsem0, gsem1, ssem0, ssem1):
    gbuf = (gbuf0, gbuf1)
    gsem = (gsem0, gsem1)
    ssem = (ssem0, ssem1)
    c = lax.axis_index("c")
    s = lax.axis_index("s")
    w = c * NS + s
    pltpu.sync_copy(rows_hbm.at[w], rowsv)
    pltpu.sync_copy(cols_hbm.at[w], colsv)
    base = s * RPT
    pltpu.sync_copy(zeros_hbm.at[pl.ds(base, RPT)], acc.at[pl.ds(base, RPT)])
    plsc.subcore_barrier()

    def chunk_body(j, b):
        nb = 1 - b
        pltpu.make_async_copy(g_hbm.at[rowsv.at[pl.ds(j * B, B)]],
                              gbuf[b], gsem[b]).wait()

        @pl.when(j >= 1)
        def _():
            pltpu.make_async_copy(gbuf[nb], acc.at[colsv.at[j - 1]],
                                  ssem[nb]).wait()

        @pl.when(j + 1 < CHUNKS)
        def _():
            pltpu.async_copy(g_hbm.at[rowsv.at[pl.ds((j + 1) * B, B)]],
                             gbuf[nb], gsem[nb])

        pltpu.async_copy(gbuf[b], acc.at[colsv.at[j]], ssem[b], add=True)

    pltpu.async_copy(g_hbm.at[rowsv.at[pl.ds(0, B)]], gbuf[0], gsem[0])

    def outer(i, carry):
        chunk_body(2 * i, 0)
        chunk_body(2 * i + 1, 1)
        return carry

    lax.fori_loop(0, (CHUNKS - 1) // 2, outer, 0)
    chunk_body(CHUNKS - 1, (CHUNKS - 1) % 2)
    lastb = (CHUNKS - 1) % 2
    pltpu.make_async_copy(gbuf[lastb], acc.at[colsv.at[CHUNKS - 1]],
                          ssem[lastb]).wait()
    plsc.subcore_barrier()

    @pl.when(s < NS - 1)
    def _():
        pltpu.sync_copy(acc.at[pl.ds(base, RPT)], out_hbm.at[c, pl.ds(base, RPT)])

    @pl.when(s == NS - 1)
    def _():
        pltpu.sync_copy(acc.at[pl.ds(base, RPT_LAST)],
                        out_hbm.at[c, pl.ds(base, RPT_LAST)])


ROWS_BLK = 2000


def _mmraw_body(x_ref, w_ref, h_ref):
    h_ref[...] = jnp.dot(x_ref[...], w_ref[...],
                         preferred_element_type=jnp.float32)


def _scale_body(degp_ref, h_ref, g_ref, dinv_ref):
    deg = degp_ref[0, :, 0:1] + degp_ref[1, :, 0:1]
    dinv = lax.rsqrt(deg)
    g_ref[...] = h_ref[...] * dinv
    dinv_ref[...] = dinv


def _mid_body(sp_ref, g1_ref, dinv_ref, b1_ref, w2_ref, g2_ref):
    dinv = dinv_ref[...]
    z1 = jnp.maximum(dinv * (sp_ref[0] + sp_ref[1] + g1_ref[...]) + b1_ref[...], 0.0)
    g2_ref[...] = jnp.dot(z1, w2_ref[...], preferred_element_type=jnp.float32) * dinv


def _fin_body(sp_ref, g2_ref, dinv_ref, b2_ref, x_ref, y_ref):
    dinv = dinv_ref[...]
    out = dinv * (sp_ref[0] + sp_ref[1] + g2_ref[...]) + b2_ref[...] + x_ref[...]
    y_ref[...] = jnp.maximum(out, 0.0)


_mmraw_call = pl.pallas_call(
    _mmraw_body,
    grid=(N // ROWS_BLK,),
    in_specs=[
        pl.BlockSpec((ROWS_BLK, D), lambda i: (i, 0)),
        pl.BlockSpec((D, D), lambda i: (0, 0)),
    ],
    out_specs=pl.BlockSpec((ROWS_BLK, D), lambda i: (i, 0)),
    out_shape=jax.ShapeDtypeStruct((N, D), jnp.float32),
)

_scale_call = pl.pallas_call(
    _scale_body,
    grid=(N // ROWS_BLK,),
    in_specs=[
        pl.BlockSpec((NC, ROWS_BLK, DW), lambda i: (0, i, 0)),
        pl.BlockSpec((ROWS_BLK, D), lambda i: (i, 0)),
    ],
    out_specs=[
        pl.BlockSpec((ROWS_BLK, D), lambda i: (i, 0)),
        pl.BlockSpec((ROWS_BLK, 1), lambda i: (i, 0)),
    ],
    out_shape=[
        jax.ShapeDtypeStruct((N, D), jnp.float32),
        jax.ShapeDtypeStruct((N, 1), jnp.float32),
    ],
)

_mid_call = pl.pallas_call(
    _mid_body,
    grid=(N // ROWS_BLK,),
    in_specs=[
        pl.BlockSpec((NC, ROWS_BLK, D), lambda i: (0, i, 0)),
        pl.BlockSpec((ROWS_BLK, D), lambda i: (i, 0)),
        pl.BlockSpec((ROWS_BLK, 1), lambda i: (i, 0)),
        pl.BlockSpec((1, D), lambda i: (0, 0)),
        pl.BlockSpec((D, D), lambda i: (0, 0)),
    ],
    out_specs=pl.BlockSpec((ROWS_BLK, D), lambda i: (i, 0)),
    out_shape=jax.ShapeDtypeStruct((N, D), jnp.float32),
)

_fin_call = pl.pallas_call(
    _fin_body,
    grid=(N // ROWS_BLK,),
    in_specs=[
        pl.BlockSpec((NC, ROWS_BLK, D), lambda i: (0, i, 0)),
        pl.BlockSpec((ROWS_BLK, D), lambda i: (i, 0)),
        pl.BlockSpec((ROWS_BLK, 1), lambda i: (i, 0)),
        pl.BlockSpec((1, D), lambda i: (0, 0)),
        pl.BlockSpec((ROWS_BLK, D), lambda i: (i, 0)),
    ],
    out_specs=pl.BlockSpec((ROWS_BLK, D), lambda i: (i, 0)),
    out_shape=jax.ShapeDtypeStruct((N, D), jnp.float32),
)


def kernel(x, edge_index, W1, b1, W2, b2):
    rows = edge_index[0].reshape(NW, EPW)
    cols = edge_index[1].reshape(NW, CHUNKS, B)
    zeros_d = jnp.zeros((NPAD, D), jnp.float32)
    zeros_w = jnp.zeros((NPAD, DW), jnp.float32)
    ones_w = jnp.ones((B, DW), jnp.float32)

    degp = _deg_kernel(cols, ones_w, zeros_w)
    h1 = _mmraw_call(x, W1)
    g1, dinv = _scale_call(degp, h1)
    s1 = _agg_kernel(g1, rows, cols, zeros_d)
    g2 = _mid_call(s1, g1, dinv, b1.reshape(1, D), W2)
    s2 = _agg_kernel(g2, rows, cols, zeros_d)
    return _fin_call(s2, g2, dinv, b2.reshape(1, D), x)


# speedup vs baseline: 1.0030x; 1.0030x over previous
import functools

import jax
import jax.numpy as jnp
from jax import lax
from jax.experimental import pallas as pl
from jax.experimental.pallas import tpu as pltpu
from jax.experimental.pallas import tpu_sc as plsc

N = 10000
E = 320000
D = 128
NC = 2
NS = 16
NW = NC * NS
B = 80
CHUNKS = 125
EPW = CHUNKS * B
NPAD = 10112
RPT = NPAD // NS
RPT_LAST = N - (NS - 1) * RPT
DW = 16

_mesh = plsc.VectorSubcoreMesh(core_axis_name="c", subcore_axis_name="s")


@functools.partial(
    pl.kernel,
    mesh=_mesh,
    out_type=jax.ShapeDtypeStruct((NC, N, DW), jnp.float32),
    scratch_types=[
        pltpu.VMEM((CHUNKS, B), jnp.int32),
        pltpu.VMEM((B, DW), jnp.float32),
        pltpu.VMEM_SHARED((NPAD, DW), jnp.float32),
        pltpu.SemaphoreType.DMA,
        pltpu.SemaphoreType.DMA,
    ],
)
def _deg_kernel(cols_hbm, ones_hbm, zeros_hbm, out_hbm, colv, onesv, acc,
                ssem0, ssem1):
    ssem = (ssem0, ssem1)
    c = lax.axis_index("c")
    s = lax.axis_index("s")
    w = c * NS + s
    pltpu.sync_copy(cols_hbm.at[w], colv)
    pltpu.sync_copy(ones_hbm, onesv)
    base = s * RPT
    pltpu.sync_copy(zeros_hbm.at[pl.ds(base, RPT)], acc.at[pl.ds(base, RPT)])
    plsc.subcore_barrier()

    def body(j, b):
        @pl.when(j >= 2)
        def _():
            pltpu.make_async_copy(onesv, acc.at[colv.at[j - 2]], ssem[b]).wait()

        pltpu.async_copy(onesv, acc.at[colv.at[j]], ssem[b], add=True)

    def outer(i, carry):
        body(2 * i, 0)
        body(2 * i + 1, 1)
        return carry

    lax.fori_loop(0, (CHUNKS - 1) // 2, outer, 0)
    body(CHUNKS - 1, (CHUNKS - 1) % 2)
    pltpu.make_async_copy(onesv, acc.at[colv.at[CHUNKS - 2]],
                          ssem[(CHUNKS - 2) % 2]).wait()
    pltpu.make_async_copy(onesv, acc.at[colv.at[CHUNKS - 1]],
                          ssem[(CHUNKS - 1) % 2]).wait()
    plsc.subcore_barrier()

    @pl.when(s < NS - 1)
    def _():
        pltpu.sync_copy(acc.at[pl.ds(base, RPT)], out_hbm.at[c, pl.ds(base, RPT)])

    @pl.when(s == NS - 1)
    def _():
        pltpu.sync_copy(acc.at[pl.ds(base, RPT_LAST)],
                        out_hbm.at[c, pl.ds(base, RPT_LAST)])


@functools.partial(
    pl.kernel,
    mesh=_mesh,
    out_type=jax.ShapeDtypeStruct((NC, N, D), jnp.float32),
    scratch_types=[
        pltpu.VMEM((CHUNKS * B,), jnp.int32),
        pltpu.VMEM((CHUNKS, B), jnp.int32),
        pltpu.VMEM_SHARED((NPAD, D), jnp.float32),
        pltpu.VMEM((B, D), jnp.float32),
        pltpu.VMEM((B, D), jnp.float32),
        pltpu.SemaphoreType.DMA,
        pltpu.SemaphoreType.DMA,
        pltpu.SemaphoreType.DMA,
        pltpu.SemaphoreType.DMA,
    ],
)
def _agg_kernel(g_hbm, rows_hbm, cols_hbm, zeros_hbm, out_hbm,
                rowsv, colsv, acc, gbuf0, gbuf1,
                gsem0, gsem1, ssem0, ssem1):
    gbuf = (gbuf0, gbuf1)
    gsem = (gsem0, gsem1)
    ssem = (ssem0, ssem1)
    c = lax.axis_index("c")
    s = lax.axis_index("s")
    w = c * NS + s
    pltpu.sync_copy(rows_hbm.at[w], rowsv)
    pltpu.sync_copy(cols_hbm.at[w], colsv)
    base = s * RPT
    pltpu.sync_copy(zeros_hbm.at[pl.ds(base, RPT)], acc.at[pl.ds(base, RPT)])
    plsc.subcore_barrier()

    def chunk_body(j, b):
        nb = 1 - b
        pltpu.make_async_copy(g_hbm.at[rowsv.at[pl.ds(j * B, B)]],
                              gbuf[b], gsem[b]).wait()

        @pl.when(j >= 1)
        def _():
            pltpu.make_async_copy(gbuf[nb], acc.at[colsv.at[j - 1]],
                                  ssem[nb]).wait()

        @pl.when(j + 1 < CHUNKS)
        def _():
            pltpu.async_copy(g_hbm.at[rowsv.at[pl.ds((j + 1) * B, B)]],
                             gbuf[nb], gsem[nb])

        pltpu.async_copy(gbuf[b], acc.at[colsv.at[j]], ssem[b], add=True)

    pltpu.async_copy(g_hbm.at[rowsv.at[pl.ds(0, B)]], gbuf[0], gsem[0])

    def outer(i, carry):
        chunk_body(2 * i, 0)
        chunk_body(2 * i + 1, 1)
        return carry

    lax.fori_loop(0, (CHUNKS - 1) // 2, outer, 0)
    chunk_body(CHUNKS - 1, (CHUNKS - 1) % 2)
    lastb = (CHUNKS - 1) % 2
    pltpu.make_async_copy(gbuf[lastb], acc.at[colsv.at[CHUNKS - 1]],
                          ssem[lastb]).wait()
    plsc.subcore_barrier()

    @pl.when(s < NS - 1)
    def _():
        pltpu.sync_copy(acc.at[pl.ds(base, RPT)], out_hbm.at[c, pl.ds(base, RPT)])

    @pl.when(s == NS - 1)
    def _():
        pltpu.sync_copy(acc.at[pl.ds(base, RPT_LAST)],
                        out_hbm.at[c, pl.ds(base, RPT_LAST)])


ROWS_BLK = 2000


def _mm1_body(degp_ref, x_ref, w_ref, g_ref, dinv_ref):
    deg = degp_ref[0, :, 0:1] + degp_ref[1, :, 0:1]
    dinv = lax.rsqrt(deg)
    h = jnp.dot(x_ref[...], w_ref[...], preferred_element_type=jnp.float32)
    g_ref[...] = h * dinv
    dinv_ref[...] = dinv


def _mid_body(sp_ref, g1_ref, dinv_ref, b1_ref, w2_ref, g2_ref):
    dinv = dinv_ref[...]
    z1 = jnp.maximum(dinv * (sp_ref[0] + sp_ref[1] + g1_ref[...]) + b1_ref[...], 0.0)
    g2_ref[...] = jnp.dot(z1, w2_ref[...], preferred_element_type=jnp.float32) * dinv


def _fin_body(sp_ref, g2_ref, dinv_ref, b2_ref, x_ref, y_ref):
    dinv = dinv_ref[...]
    out = dinv * (sp_ref[0] + sp_ref[1] + g2_ref[...]) + b2_ref[...] + x_ref[...]
    y_ref[...] = jnp.maximum(out, 0.0)


_mm1_call = pl.pallas_call(
    _mm1_body,
    grid=(N // ROWS_BLK,),
    in_specs=[
        pl.BlockSpec((NC, ROWS_BLK, DW), lambda i: (0, i, 0)),
        pl.BlockSpec((ROWS_BLK, D), lambda i: (i, 0)),
        pl.BlockSpec((D, D), lambda i: (0, 0)),
    ],
    out_specs=[
        pl.BlockSpec((ROWS_BLK, D), lambda i: (i, 0)),
        pl.BlockSpec((ROWS_BLK, 1), lambda i: (i, 0)),
    ],
    out_shape=[
        jax.ShapeDtypeStruct((N, D), jnp.float32),
        jax.ShapeDtypeStruct((N, 1), jnp.float32),
    ],
)

_mid_call = pl.pallas_call(
    _mid_body,
    grid=(N // ROWS_BLK,),
    in_specs=[
        pl.BlockSpec((NC, ROWS_BLK, D), lambda i: (0, i, 0)),
        pl.BlockSpec((ROWS_BLK, D), lambda i: (i, 0)),
        pl.BlockSpec((ROWS_BLK, 1), lambda i: (i, 0)),
        pl.BlockSpec((1, D), lambda i: (0, 0)),
        pl.BlockSpec((D, D), lambda i: (0, 0)),
    ],
    out_specs=pl.BlockSpec((ROWS_BLK, D), lambda i: (i, 0)),
    out_shape=jax.ShapeDtypeStruct((N, D), jnp.float32),
)

_fin_call = pl.pallas_call(
    _fin_body,
    grid=(N // ROWS_BLK,),
    in_specs=[
        pl.BlockSpec((NC, ROWS_BLK, D), lambda i: (0, i, 0)),
        pl.BlockSpec((ROWS_BLK, D), lambda i: (i, 0)),
        pl.BlockSpec((ROWS_BLK, 1), lambda i: (i, 0)),
        pl.BlockSpec((1, D), lambda i: (0, 0)),
        pl.BlockSpec((ROWS_BLK, D), lambda i: (i, 0)),
    ],
    out_specs=pl.BlockSpec((ROWS_BLK, D), lambda i: (i, 0)),
    out_shape=jax.ShapeDtypeStruct((N, D), jnp.float32),
)


def kernel(x, edge_index, W1, b1, W2, b2):
    rows = edge_index[0].reshape(NW, EPW)
    cols = edge_index[1].reshape(NW, CHUNKS, B)
    zeros_d = jnp.zeros((NPAD, D), jnp.float32)
    zeros_w = jnp.zeros((NPAD, DW), jnp.float32)
    ones_w = jnp.ones((B, DW), jnp.float32)

    degp = _deg_kernel(cols, ones_w, zeros_w)
    g1, dinv = _mm1_call(degp, x, W1)
    s1 = _agg_kernel(g1, rows, cols, zeros_d)
    g2 = _mid_call(s1, g1, dinv, b1.reshape(1, D), W2)
    s2 = _agg_kernel(g2, rows, cols, zeros_d)
    return _fin_call(s2, g2, dinv, b2.reshape(1, D), x)
